# Initial kernel scaffold; baseline (speedup 1.0000x reference)
import jax
import jax.numpy as jnp
from jax.experimental import pallas as pl


def kernel(displacement, coords, properties, volume, bc_values, domain_id, bc_type, edge_index):
    raise NotImplementedError("write your pallas kernel here")


# speedup vs baseline: 190.7995x; 190.7995x over previous
import dataclasses
import functools
import math

import jax
import jax.numpy as jnp
from jax import lax
from jax.experimental import pallas as pl
from jax.experimental.pallas import tpu as pltpu
from jax.experimental.pallas import tpu_sc as plsc

H_SMOOTH = 2.0
_SIGMA3 = 1.0 / math.pi
_KC = -_SIGMA3 / (H_SMOOTH ** 4)
_INV_H = 1.0 / H_SMOOTH

_NC = 2
_NS = 16
_NW = _NC * _NS
_L = 16
_CHUNK = 128


def _rsqrt_nr(x):
    i = plsc.bitcast(x, jnp.int32)
    i = jnp.int32(0x5F3759DF) - lax.shift_right_logical(i, 1)
    y = plsc.bitcast(i, jnp.float32)
    xh = 0.5 * x
    for _ in range(3):
        y = y * (1.5 - xh * y * y)
    return y


def _edge_kernel_body(tab_hbm, ii_hbm, jj_hbm, out_hbm,
                      ii_v, jj_v, irows, jrows, contrib, accum,
                      *, n_pad, chunks_per_tile):
    c = lax.axis_index("c")
    s = lax.axis_index("s")
    npt = n_pad // _NS

    @pl.loop(0, _CHUNK)
    def _(r):
        contrib[r, :] = jnp.zeros((_L,), jnp.float32)

    nbase = s * npt

    @pl.loop(0, (npt + _CHUNK - 1) // _CHUNK)
    def _(z):
        off = jnp.minimum(z * _CHUNK, npt - _CHUNK)
        pltpu.sync_copy(contrib, accum.at[pl.ds(nbase + off, _CHUNK)])

    plsc.subcore_barrier()

    w = c * _NS + s
    ebase = w * (chunks_per_tile * _CHUNK)
    lanes = lax.iota(jnp.int32, _L)

    @pl.loop(0, chunks_per_tile)
    def _(k):
        eb = ebase + k * _CHUNK
        pltpu.sync_copy(ii_hbm.at[pl.ds(eb, _CHUNK)], ii_v)
        pltpu.sync_copy(jj_hbm.at[pl.ds(eb, _CHUNK)], jj_v)
        pltpu.sync_copy(tab_hbm.at[ii_v], irows)
        pltpu.sync_copy(tab_hbm.at[jj_v], jrows)

        @pl.loop(0, _CHUNK // _L)
        def _(g):
            rbase = g * _L + lanes

            def ld(ref, col):
                cidx = jnp.full((_L,), col, jnp.int32)
                return plsc.load_gather(ref, [rbase, cidx])

            vj = ld(jrows, 12)

            def st(col, val):
                cidx = jnp.full((_L,), col, jnp.int32)
                plsc.store_scatter(contrib, [rbase, cidx], val)

            for t in (0, 1):
                o = 6 * t
                rx = ld(jrows, o + 0) - ld(irows, o + 0)
                ry = ld(jrows, o + 1) - ld(irows, o + 1)
                rz = ld(jrows, o + 2) - ld(irows, o + 2)
                a0 = ld(jrows, o + 3) - ld(irows, o + 3)
                a1 = ld(jrows, o + 4) - ld(irows, o + 4)
                a2 = ld(jrows, o + 5) - ld(irows, o + 5)
                r2 = jnp.maximum(rx * rx + ry * ry + rz * rz, 1e-16)
                inv_s = _rsqrt_nr(r2)
                rs = r2 * inv_s
                q = rs * _INV_H
                b1 = q * (2.25 * q - 3.0)
                tq = 2.0 - q
                b2 = -0.75 * tq * tq
                dwdq = jnp.where(q < 1.0, b1,
                                 jnp.where(q < 2.0, b2, jnp.float32(0.0)))
                kf = (_KC * vj) * (dwdq * inv_s)
                b0v = kf * rx
                b1v = kf * ry
                b2v = kf * rz
                st(8 * t + 0, a0 * b0v)
                st(8 * t + 1, a1 * b1v)
                st(8 * t + 2, a2 * b2v)
                st(8 * t + 3, a0 * b1v + a1 * b0v)
                st(8 * t + 4, a1 * b2v + a2 * b1v)
                st(8 * t + 5, a2 * b0v + a0 * b2v)

        pltpu.sync_copy(contrib, accum.at[ii_v], add=True)

    plsc.subcore_barrier()
    pltpu.sync_copy(accum.at[pl.ds(nbase, npt)],
                    out_hbm.at[c, pl.ds(nbase, npt)])


def _sc_edge_pass(table, ii, jj, n_pad, chunks_per_tile):
    mesh = plsc.VectorSubcoreMesh(core_axis_name="c", subcore_axis_name="s",
                                  num_cores=_NC, num_subcores=_NS)
    body = functools.partial(_edge_kernel_body, n_pad=n_pad,
                             chunks_per_tile=chunks_per_tile)
    cp = pltpu.CompilerParams()
    for fld, val in (("needs_layout_passes", False),
                     ("use_tc_tiling_on_sc", False)):
        if fld in pltpu.CompilerParams.__dataclass_fields__:
            cp = dataclasses.replace(cp, **{fld: val})
    return pl.kernel(
        body,
        compiler_params=cp,
        out_type=jax.ShapeDtypeStruct((_NC, n_pad, 16), jnp.float32),
        mesh=mesh,
        scratch_types=[
            pltpu.VMEM((_CHUNK,), jnp.int32),
            pltpu.VMEM((_CHUNK,), jnp.int32),
            pltpu.VMEM((_CHUNK, 16), jnp.float32),
            pltpu.VMEM((_CHUNK, 16), jnp.float32),
            pltpu.VMEM((_CHUNK, 16), jnp.float32),
            pltpu.VMEM_SHARED((n_pad, 16), jnp.float32),
        ],
    )(table, ii, jj)


def _node_kernel_body(s0, s1, p, v, u, bv, bt, out):
    vol = v[:, :]
    pi_int = jnp.float32(0.0)
    pi_ext = jnp.float32(0.0)
    for t in (0, 1):
        sv = [s0[6 * t + k] + s1[6 * t + k] for k in range(6)]
        e_mod = p[2 * t]
        nu = p[2 * t + 1]
        one_m2nu = 1.0 - 2.0 * nu
        lam = e_mod / ((1.0 + nu) * one_m2nu)
        aa = nu * lam
        bb = one_m2nu * lam
        tr = sv[0] + sv[1] + sv[2]
        dc = (aa * tr * tr
              + bb * (sv[0] * sv[0] + sv[1] * sv[1] + sv[2] * sv[2])
              + (0.5 * bb) * (sv[3] * sv[3] + sv[4] * sv[4] + sv[5] * sv[5]))
        pi_int = pi_int + 0.5 * jnp.maximum(jnp.sum(dc * vol), 0.0)
        dot = (bv[3 * t] * u[3 * t] + bv[3 * t + 1] * u[3 * t + 1]
               + bv[3 * t + 2] * u[3 * t + 2])
        masked = jnp.where(bt[t] == 2.0, dot, jnp.float32(0.0))
        pi_ext = pi_ext - jnp.sum(masked * vol)
    out[0, 0] = 0.5 * pi_int + 0.5 * pi_ext


def _node_pass(s0c, s1c, pc, vc, uc, bvc, btc):
    return pl.pallas_call(
        _node_kernel_body,
        out_shape=jax.ShapeDtypeStruct((1, 1), jnp.float32),
        out_specs=pl.BlockSpec(memory_space=pltpu.SMEM),
    )(s0c, s1c, pc, vc, uc, bvc, btc)


def kernel(displacement, coords, properties, volume, bc_values, domain_id,
           bc_type, edge_index):
    n = displacement.shape[0]
    e = edge_index.shape[1]
    f32 = jnp.float32

    table = jnp.concatenate([
        coords[:, 0, :], displacement[:, 0, :],
        coords[:, 1, :], displacement[:, 1, :],
        volume.astype(f32), jnp.zeros((n, 3), f32),
    ], axis=1)

    chunks_per_tile = -(-e // (_NW * _CHUNK))
    e_pad = chunks_per_tile * _NW * _CHUNK
    ii = edge_index[0].astype(jnp.int32)
    jj = edge_index[1].astype(jnp.int32)
    if e_pad != e:
        zpad = jnp.zeros((e_pad - e,), jnp.int32)
        ii = jnp.concatenate([ii, zpad])
        jj = jnp.concatenate([jj, zpad])

    n_pad = -(-n // (_NS * 8)) * (_NS * 8)
    if n_pad != n:
        table = jnp.concatenate([table, jnp.zeros((n_pad - n, 16), f32)])

    partial = _sc_edge_pass(table, ii, jj, n_pad, chunks_per_tile)

    rows = -(-n // 128)
    np2 = rows * 128

    def colpack(a2d, cols):
        x = jnp.stack([a2d[:n, k] for k in cols], 0)
        x = jnp.pad(x, ((0, 0), (0, np2 - n)))
        return x.reshape(len(cols), rows, 128)

    vo_cols = [0, 1, 2, 3, 4, 5, 8, 9, 10, 11, 12, 13]
    s0c = colpack(partial[0], vo_cols)
    s1c = colpack(partial[1], vo_cols)
    pc = colpack(properties.reshape(n, 4), [0, 1, 2, 3])
    vc = colpack(volume.astype(f32), [0])[0]
    uc = colpack(displacement.reshape(n, 6), list(range(6)))
    bvc = colpack(bc_values.reshape(n, 6), list(range(6)))
    btc = colpack(bc_type.astype(f32), [0, 1])

    res = _node_pass(s0c, s1c, pc, vc, uc, bvc, btc)
    return res[0, 0]


# speedup vs baseline: 292.6244x; 1.5337x over previous
import dataclasses
import functools
import math

import jax
import jax.numpy as jnp
from jax import lax
from jax.experimental import pallas as pl
from jax.experimental.pallas import tpu as pltpu
from jax.experimental.pallas import tpu_sc as plsc

H_SMOOTH = 2.0
_SIGMA3 = 1.0 / math.pi
_KC = -_SIGMA3 / (H_SMOOTH ** 4)
_INV_H = 1.0 / H_SMOOTH

_NC = 2
_NS = 16
_NW = _NC * _NS
_L = 16
_CHUNK = 128
_IBLK = 16
_SLOTS = 4


def _rsqrt_nr(x):
    i = plsc.bitcast(x, jnp.int32)
    i = jnp.int32(0x5F3759DF) - lax.shift_right_logical(i, 1)
    y = plsc.bitcast(i, jnp.float32)
    xh = 0.5 * x
    for _ in range(3):
        y = y * (1.5 - xh * y * y)
    return y


def _compute_chunk(irows, jrows, contrib, lanes):
    @pl.loop(0, _CHUNK // _L)
    def _(g):
        rbase = g * _L + lanes

        def ld(ref, col):
            cidx = jnp.full((_L,), col, jnp.int32)
            return plsc.load_gather(ref, [rbase, cidx])

        def st(col, val):
            cidx = jnp.full((_L,), col, jnp.int32)
            plsc.store_scatter(contrib, [rbase, cidx], val)

        vj = ld(jrows, 12)
        for t in (0, 1):
            o = 6 * t
            rx = ld(jrows, o + 0) - ld(irows, o + 0)
            ry = ld(jrows, o + 1) - ld(irows, o + 1)
            rz = ld(jrows, o + 2) - ld(irows, o + 2)
            a0 = ld(jrows, o + 3) - ld(irows, o + 3)
            a1 = ld(jrows, o + 4) - ld(irows, o + 4)
            a2 = ld(jrows, o + 5) - ld(irows, o + 5)
            r2 = jnp.maximum(rx * rx + ry * ry + rz * rz, 1e-16)
            inv_s = _rsqrt_nr(r2)
            rs = r2 * inv_s
            q = rs * _INV_H
            b1 = q * (2.25 * q - 3.0)
            tq = 2.0 - q
            b2 = -0.75 * tq * tq
            dwdq = jnp.where(q < 1.0, b1,
                             jnp.where(q < 2.0, b2, jnp.float32(0.0)))
            kf = (_KC * vj) * (dwdq * inv_s)
            b0v = kf * rx
            b1v = kf * ry
            b2v = kf * rz
            st(8 * t + 0, a0 * b0v)
            st(8 * t + 1, a1 * b1v)
            st(8 * t + 2, a2 * b2v)
            st(8 * t + 3, a0 * b1v + a1 * b0v)
            st(8 * t + 4, a1 * b2v + a2 * b1v)
            st(8 * t + 5, a2 * b0v + a0 * b2v)


def _edge_kernel_body(tab_hbm, ii_hbm, jj_hbm, out_hbm,
                      ibuf, jbuf, irows, jrows, contrib, accum,
                      sem_i, sem_g, sem_s,
                      *, n_pad, chunks_per_tile):
    c = lax.axis_index("c")
    s = lax.axis_index("s")
    npt = n_pad // _NS
    lanes = lax.iota(jnp.int32, _L)

    @pl.loop(0, _SLOTS * _CHUNK)
    def _(r):
        contrib[r // _CHUNK, r % _CHUNK, :] = jnp.zeros((_L,), jnp.float32)

    nbase = s * npt

    @pl.loop(0, (npt + _CHUNK - 1) // _CHUNK)
    def _(z):
        off = jnp.minimum(z * _CHUNK, npt - _CHUNK)
        pltpu.sync_copy(contrib.at[0], accum.at[pl.ds(nbase + off, _CHUNK)])

    plsc.subcore_barrier()

    w = c * _NS + s
    rowbase = w * chunks_per_tile

    @pl.loop(0, chunks_per_tile // _IBLK)
    def _(m):
        blk = rowbase + m * _IBLK
        pltpu.sync_copy(ii_hbm.at[pl.ds(blk, _IBLK)], ibuf)
        pltpu.sync_copy(jj_hbm.at[pl.ds(blk, _IBLK)], jbuf)

        gath = [None] * _IBLK
        scat = [None] * _IBLK

        def fire(k):
            sl = k % _SLOTS
            gath[k] = (
                pltpu.async_copy(tab_hbm.at[ibuf.at[k]], irows.at[sl],
                                 sem_g[sl]),
                pltpu.async_copy(tab_hbm.at[jbuf.at[k]], jrows.at[sl],
                                 sem_g[sl]),
            )

        for k in range(_SLOTS - 1):
            fire(k)
        for k in range(_IBLK):
            sl = k % _SLOTS
            if k + _SLOTS - 1 < _IBLK:
                fire(k + _SLOTS - 1)
            d1, d2 = gath[k]
            d1.wait()
            d2.wait()
            if k >= _SLOTS:
                scat[k - _SLOTS].wait()
            _compute_chunk(irows.at[sl], jrows.at[sl], contrib.at[sl], lanes)
            scat[k] = pltpu.async_copy(contrib.at[sl], accum.at[ibuf.at[k]],
                                       sem_s[sl], add=True)
        for k in range(_IBLK - _SLOTS, _IBLK):
            scat[k].wait()

    plsc.subcore_barrier()
    pltpu.sync_copy(accum.at[pl.ds(nbase, npt)],
                    out_hbm.at[c, pl.ds(nbase, npt)])


def _sc_edge_pass(table, ii, jj, n_pad, chunks_per_tile):
    mesh = plsc.VectorSubcoreMesh(core_axis_name="c", subcore_axis_name="s",
                                  num_cores=_NC, num_subcores=_NS)
    body = functools.partial(_edge_kernel_body, n_pad=n_pad,
                             chunks_per_tile=chunks_per_tile)
    cp = pltpu.CompilerParams()
    for fld, val in (("needs_layout_passes", False),
                     ("use_tc_tiling_on_sc", False)):
        if fld in pltpu.CompilerParams.__dataclass_fields__:
            cp = dataclasses.replace(cp, **{fld: val})

    def wrapped(tab_hbm, ii_hbm, jj_hbm, out_hbm, ibuf, jbuf, irows, jrows,
                contrib, accum, sem_i, sg0, sg1, sg2, sg3, ss0, ss1, ss2,
                ss3):
        _edge_kernel_body(tab_hbm, ii_hbm, jj_hbm, out_hbm, ibuf, jbuf,
                          irows, jrows, contrib, accum, sem_i,
                          [sg0, sg1, sg2, sg3], [ss0, ss1, ss2, ss3],
                          n_pad=n_pad, chunks_per_tile=chunks_per_tile)

    return pl.kernel(
        wrapped,
        out_type=jax.ShapeDtypeStruct((_NC, n_pad, 16), jnp.float32),
        mesh=mesh,
        compiler_params=cp,
        scratch_types=[
            pltpu.VMEM((_IBLK, _CHUNK), jnp.int32),
            pltpu.VMEM((_IBLK, _CHUNK), jnp.int32),
            pltpu.VMEM((_SLOTS, _CHUNK, 16), jnp.float32),
            pltpu.VMEM((_SLOTS, _CHUNK, 16), jnp.float32),
            pltpu.VMEM((_SLOTS, _CHUNK, 16), jnp.float32),
            pltpu.VMEM_SHARED((n_pad, 16), jnp.float32),
        ] + [pltpu.SemaphoreType.DMA] * 9,
    )(table, ii, jj)


def _node_kernel_body(s0, s1, p, v, u, bv, bt, out):
    vol = v[:, :]
    pi_int = jnp.float32(0.0)
    pi_ext = jnp.float32(0.0)
    for t in (0, 1):
        sv = [s0[6 * t + k] + s1[6 * t + k] for k in range(6)]
        e_mod = p[2 * t]
        nu = p[2 * t + 1]
        one_m2nu = 1.0 - 2.0 * nu
        lam = e_mod / ((1.0 + nu) * one_m2nu)
        aa = nu * lam
        bb = one_m2nu * lam
        tr = sv[0] + sv[1] + sv[2]
        dc = (aa * tr * tr
              + bb * (sv[0] * sv[0] + sv[1] * sv[1] + sv[2] * sv[2])
              + (0.5 * bb) * (sv[3] * sv[3] + sv[4] * sv[4] + sv[5] * sv[5]))
        pi_int = pi_int + 0.5 * jnp.maximum(jnp.sum(dc * vol), 0.0)
        dot = (bv[3 * t] * u[3 * t] + bv[3 * t + 1] * u[3 * t + 1]
               + bv[3 * t + 2] * u[3 * t + 2])
        masked = jnp.where(bt[t] == 2.0, dot, jnp.float32(0.0))
        pi_ext = pi_ext - jnp.sum(masked * vol)
    out[0, 0] = 0.5 * pi_int + 0.5 * pi_ext


def _node_pass(s0c, s1c, pc, vc, uc, bvc, btc):
    return pl.pallas_call(
        _node_kernel_body,
        out_shape=jax.ShapeDtypeStruct((1, 1), jnp.float32),
        out_specs=pl.BlockSpec(memory_space=pltpu.SMEM),
    )(s0c, s1c, pc, vc, uc, bvc, btc)


def kernel(displacement, coords, properties, volume, bc_values, domain_id,
           bc_type, edge_index):
    n = displacement.shape[0]
    e = edge_index.shape[1]
    f32 = jnp.float32

    table = jnp.concatenate([
        coords[:, 0, :], displacement[:, 0, :],
        coords[:, 1, :], displacement[:, 1, :],
        volume.astype(f32), jnp.zeros((n, 3), f32),
    ], axis=1)

    grain = _NW * _CHUNK * _IBLK
    e_pad = -(-e // grain) * grain
    chunks_per_tile = e_pad // (_NW * _CHUNK)
    ii = edge_index[0].astype(jnp.int32)
    jj = edge_index[1].astype(jnp.int32)
    if e_pad != e:
        zpad = jnp.zeros((e_pad - e,), jnp.int32)
        ii = jnp.concatenate([ii, zpad])
        jj = jnp.concatenate([jj, zpad])
    ii = ii.reshape(e_pad // _CHUNK, _CHUNK)
    jj = jj.reshape(e_pad // _CHUNK, _CHUNK)

    n_pad = -(-n // (_NS * 8)) * (_NS * 8)
    if n_pad != n:
        table = jnp.concatenate([table, jnp.zeros((n_pad - n, 16), f32)])

    partial = _sc_edge_pass(table, ii, jj, n_pad, chunks_per_tile)

    rows = -(-n // 128)
    np2 = rows * 128

    def colpack(a2d, cols):
        x = jnp.stack([a2d[:n, k] for k in cols], 0)
        x = jnp.pad(x, ((0, 0), (0, np2 - n)))
        return x.reshape(len(cols), rows, 128)

    vo_cols = [0, 1, 2, 3, 4, 5, 8, 9, 10, 11, 12, 13]
    s0c = colpack(partial[0], vo_cols)
    s1c = colpack(partial[1], vo_cols)
    pc = colpack(properties.reshape(n, 4), [0, 1, 2, 3])
    vc = colpack(volume.astype(f32), [0])[0]
    uc = colpack(displacement.reshape(n, 6), list(range(6)))
    bvc = colpack(bc_values.reshape(n, 6), list(range(6)))
    btc = colpack(bc_type.astype(f32), [0, 1])

    res = _node_pass(s0c, s1c, pc, vc, uc, bvc, btc)
    return res[0, 0]


# speedup vs baseline: 326.2457x; 1.1149x over previous
import dataclasses
import functools
import math

import jax
import jax.numpy as jnp
from jax import lax
from jax.experimental import pallas as pl
from jax.experimental.pallas import tpu as pltpu
from jax.experimental.pallas import tpu_sc as plsc

H_SMOOTH = 2.0
_SIGMA3 = 1.0 / math.pi
_KC = -_SIGMA3 / (H_SMOOTH ** 4)
_INV_H = 1.0 / H_SMOOTH

_NC = 2
_NS = 16
_NW = _NC * _NS
_L = 16
_CHUNK = 128
_IBLK = 16
_SLOTS = 4


def _rsqrt_nr(x):
    i = plsc.bitcast(x, jnp.int32)
    i = jnp.int32(0x5F3759DF) - lax.shift_right_logical(i, 1)
    y = plsc.bitcast(i, jnp.float32)
    xh = 0.5 * x
    for _ in range(3):
        y = y * (1.5 - xh * y * y)
    return y


def _compute_chunk(irows, jrows, contrib, lanes):
    @pl.loop(0, _CHUNK // _L, unroll=2)
    def _(g):
        rbase = g * _L + lanes

        def ld(ref, col):
            cidx = jnp.full((_L,), col, jnp.int32)
            return plsc.load_gather(ref, [rbase, cidx])

        def st(col, val):
            cidx = jnp.full((_L,), col, jnp.int32)
            plsc.store_scatter(contrib, [rbase, cidx], val)

        vj = ld(jrows, 12)
        for t in (0, 1):
            o = 6 * t
            rx = ld(jrows, o + 0) - ld(irows, o + 0)
            ry = ld(jrows, o + 1) - ld(irows, o + 1)
            rz = ld(jrows, o + 2) - ld(irows, o + 2)
            a0 = ld(jrows, o + 3) - ld(irows, o + 3)
            a1 = ld(jrows, o + 4) - ld(irows, o + 4)
            a2 = ld(jrows, o + 5) - ld(irows, o + 5)
            r2 = jnp.maximum(rx * rx + ry * ry + rz * rz, 1e-16)
            inv_s = _rsqrt_nr(r2)
            rs = r2 * inv_s
            q = rs * _INV_H
            b1 = q * (2.25 * q - 3.0)
            tq = 2.0 - q
            b2 = -0.75 * tq * tq
            dwdq = jnp.where(q < 1.0, b1,
                             jnp.where(q < 2.0, b2, jnp.float32(0.0)))
            kf = (_KC * vj) * (dwdq * inv_s)
            b0v = kf * rx
            b1v = kf * ry
            b2v = kf * rz
            st(8 * t + 0, a0 * b0v)
            st(8 * t + 1, a1 * b1v)
            st(8 * t + 2, a2 * b2v)
            st(8 * t + 3, a0 * b1v + a1 * b0v)
            st(8 * t + 4, a1 * b2v + a2 * b1v)
            st(8 * t + 5, a2 * b0v + a0 * b2v)


def _edge_kernel_body(tab_hbm, ii_hbm, jj_hbm, out_hbm,
                      ibuf, jbuf, irows, jrows, contrib, accum,
                      sem_i, sem_g, sem_s,
                      *, n_pad, chunks_per_tile):
    c = lax.axis_index("c")
    s = lax.axis_index("s")
    npt = n_pad // _NS
    lanes = lax.iota(jnp.int32, _L)

    @pl.loop(0, _SLOTS * _CHUNK)
    def _(r):
        contrib[r // _CHUNK, r % _CHUNK, :] = jnp.zeros((_L,), jnp.float32)

    nbase = s * npt

    @pl.loop(0, (npt + _CHUNK - 1) // _CHUNK)
    def _(z):
        off = jnp.minimum(z * _CHUNK, npt - _CHUNK)
        pltpu.sync_copy(contrib.at[0], accum.at[pl.ds(nbase + off, _CHUNK)])

    plsc.subcore_barrier()

    w = c * _NS + s
    rowbase = w * chunks_per_tile

    @pl.loop(0, chunks_per_tile // _IBLK)
    def _(m):
        blk = rowbase + m * _IBLK
        pltpu.sync_copy(ii_hbm.at[pl.ds(blk, _IBLK)], ibuf)
        pltpu.sync_copy(jj_hbm.at[pl.ds(blk, _IBLK)], jbuf)

        gath = [None] * _IBLK
        scat = [None] * _IBLK

        def fire(k):
            sl = k % _SLOTS
            gath[k] = (
                pltpu.async_copy(tab_hbm.at[ibuf.at[k]], irows.at[sl],
                                 sem_g[sl]),
                pltpu.async_copy(tab_hbm.at[jbuf.at[k]], jrows.at[sl],
                                 sem_g[sl]),
            )

        for k in range(_SLOTS - 1):
            fire(k)
        for k in range(_IBLK):
            sl = k % _SLOTS
            if k + _SLOTS - 1 < _IBLK:
                fire(k + _SLOTS - 1)
            d1, d2 = gath[k]
            d1.wait()
            d2.wait()
            if k >= _SLOTS:
                scat[k - _SLOTS].wait()
            _compute_chunk(irows.at[sl], jrows.at[sl], contrib.at[sl], lanes)
            scat[k] = pltpu.async_copy(contrib.at[sl], accum.at[ibuf.at[k]],
                                       sem_s[sl], add=True)
        for k in range(_IBLK - _SLOTS, _IBLK):
            scat[k].wait()

    plsc.subcore_barrier()
    pltpu.sync_copy(accum.at[pl.ds(nbase, npt)],
                    out_hbm.at[c, pl.ds(nbase, npt)])


def _sc_edge_pass(table, ii, jj, n_pad, chunks_per_tile):
    mesh = plsc.VectorSubcoreMesh(core_axis_name="c", subcore_axis_name="s",
                                  num_cores=_NC, num_subcores=_NS)
    body = functools.partial(_edge_kernel_body, n_pad=n_pad,
                             chunks_per_tile=chunks_per_tile)
    cp = pltpu.CompilerParams()
    for fld, val in (("needs_layout_passes", False),
                     ("use_tc_tiling_on_sc", False)):
        if fld in pltpu.CompilerParams.__dataclass_fields__:
            cp = dataclasses.replace(cp, **{fld: val})

    def wrapped(tab_hbm, ii_hbm, jj_hbm, out_hbm, ibuf, jbuf, irows, jrows,
                contrib, accum, sem_i, sg0, sg1, sg2, sg3, ss0, ss1, ss2,
                ss3):
        _edge_kernel_body(tab_hbm, ii_hbm, jj_hbm, out_hbm, ibuf, jbuf,
                          irows, jrows, contrib, accum, sem_i,
                          [sg0, sg1, sg2, sg3], [ss0, ss1, ss2, ss3],
                          n_pad=n_pad, chunks_per_tile=chunks_per_tile)

    return pl.kernel(
        wrapped,
        out_type=jax.ShapeDtypeStruct((_NC, n_pad, 16), jnp.float32),
        mesh=mesh,
        compiler_params=cp,
        scratch_types=[
            pltpu.VMEM((_IBLK, _CHUNK), jnp.int32),
            pltpu.VMEM((_IBLK, _CHUNK), jnp.int32),
            pltpu.VMEM((_SLOTS, _CHUNK, 16), jnp.float32),
            pltpu.VMEM((_SLOTS, _CHUNK, 16), jnp.float32),
            pltpu.VMEM((_SLOTS, _CHUNK, 16), jnp.float32),
            pltpu.VMEM_SHARED((n_pad, 16), jnp.float32),
        ] + [pltpu.SemaphoreType.DMA] * 9,
    )(table, ii, jj)


def _node_kernel_body(s0, s1, p, v, u, bv, bt, out):
    vol = v[:, :]
    pi_int = jnp.float32(0.0)
    pi_ext = jnp.float32(0.0)
    for t in (0, 1):
        sv = [s0[6 * t + k] + s1[6 * t + k] for k in range(6)]
        e_mod = p[2 * t]
        nu = p[2 * t + 1]
        one_m2nu = 1.0 - 2.0 * nu
        lam = e_mod / ((1.0 + nu) * one_m2nu)
        aa = nu * lam
        bb = one_m2nu * lam
        tr = sv[0] + sv[1] + sv[2]
        dc = (aa * tr * tr
              + bb * (sv[0] * sv[0] + sv[1] * sv[1] + sv[2] * sv[2])
              + (0.5 * bb) * (sv[3] * sv[3] + sv[4] * sv[4] + sv[5] * sv[5]))
        pi_int = pi_int + 0.5 * jnp.maximum(jnp.sum(dc * vol), 0.0)
        dot = (bv[3 * t] * u[3 * t] + bv[3 * t + 1] * u[3 * t + 1]
               + bv[3 * t + 2] * u[3 * t + 2])
        masked = jnp.where(bt[t] == 2.0, dot, jnp.float32(0.0))
        pi_ext = pi_ext - jnp.sum(masked * vol)
    out[0, 0] = 0.5 * pi_int + 0.5 * pi_ext


def _node_pass(s0c, s1c, pc, vc, uc, bvc, btc):
    return pl.pallas_call(
        _node_kernel_body,
        out_shape=jax.ShapeDtypeStruct((1, 1), jnp.float32),
        out_specs=pl.BlockSpec(memory_space=pltpu.SMEM),
    )(s0c, s1c, pc, vc, uc, bvc, btc)


def kernel(displacement, coords, properties, volume, bc_values, domain_id,
           bc_type, edge_index):
    n = displacement.shape[0]
    e = edge_index.shape[1]
    f32 = jnp.float32

    table = jnp.concatenate([
        coords[:, 0, :], displacement[:, 0, :],
        coords[:, 1, :], displacement[:, 1, :],
        volume.astype(f32), jnp.zeros((n, 3), f32),
    ], axis=1)

    grain = _NW * _CHUNK * _IBLK
    e_pad = -(-e // grain) * grain
    chunks_per_tile = e_pad // (_NW * _CHUNK)
    ii = edge_index[0].astype(jnp.int32)
    jj = edge_index[1].astype(jnp.int32)
    if e_pad != e:
        zpad = jnp.arange(e_pad - e, dtype=jnp.int32) % jnp.int32(n)
        ii = jnp.concatenate([ii, zpad])
        jj = jnp.concatenate([jj, zpad])
    ii = ii.reshape(e_pad // _CHUNK, _CHUNK)
    jj = jj.reshape(e_pad // _CHUNK, _CHUNK)

    n_pad = -(-n // (_NS * 8)) * (_NS * 8)
    if n_pad != n:
        table = jnp.concatenate([table, jnp.zeros((n_pad - n, 16), f32)])

    partial = _sc_edge_pass(table, ii, jj, n_pad, chunks_per_tile)

    rows = -(-n // 128)
    np2 = rows * 128

    def colpack(a2d, cols):
        x = jnp.stack([a2d[:n, k] for k in cols], 0)
        x = jnp.pad(x, ((0, 0), (0, np2 - n)))
        return x.reshape(len(cols), rows, 128)

    vo_cols = [0, 1, 2, 3, 4, 5, 8, 9, 10, 11, 12, 13]
    s0c = colpack(partial[0], vo_cols)
    s1c = colpack(partial[1], vo_cols)
    pc = colpack(properties.reshape(n, 4), [0, 1, 2, 3])
    vc = colpack(volume.astype(f32), [0])[0]
    uc = colpack(displacement.reshape(n, 6), list(range(6)))
    bvc = colpack(bc_values.reshape(n, 6), list(range(6)))
    btc = colpack(bc_type.astype(f32), [0, 1])

    res = _node_pass(s0c, s1c, pc, vc, uc, bvc, btc)
    return res[0, 0]


# speedup vs baseline: 467.7407x; 1.4337x over previous
import dataclasses
import functools
import math

import jax
import jax.numpy as jnp
import numpy as np
from jax import lax
from jax.experimental import pallas as pl
from jax.experimental.pallas import tpu as pltpu
from jax.experimental.pallas import tpu_sc as plsc

H_SMOOTH = 2.0
_SIGMA3 = 1.0 / math.pi
_KC = -_SIGMA3 / (H_SMOOTH ** 4)
_INV_H = 1.0 / H_SMOOTH

_NC = 2
_NS = 16
_NW = _NC * _NS
_L = 16
_CHUNK = 128
_IBLK = 16
_SLOTS = 4


def _rsqrt_nr(x):
    i = plsc.bitcast(x, jnp.int32)
    i = jnp.int32(0x5F3759DF) - lax.shift_right_logical(i, 1)
    y = plsc.bitcast(i, jnp.float32)
    xh = 0.5 * x
    for _ in range(3):
        y = y * (1.5 - xh * y * y)
    return y


def _compute_chunk(irows, jrows, contrib, lanes):
    @pl.loop(0, _CHUNK // _L, unroll=2)
    def _(g):
        rbase = g * _L + lanes

        def ld(ref, col):
            cidx = jnp.full((_L,), col, jnp.int32)
            return plsc.load_gather(ref, [rbase, cidx])

        def st(col, val):
            cidx = jnp.full((_L,), col, jnp.int32)
            plsc.store_scatter(contrib, [rbase, cidx], val)

        vj = ld(jrows, 12)
        for t in (0, 1):
            o = 6 * t
            rx = ld(jrows, o + 0) - ld(irows, o + 0)
            ry = ld(jrows, o + 1) - ld(irows, o + 1)
            rz = ld(jrows, o + 2) - ld(irows, o + 2)
            a0 = ld(jrows, o + 3) - ld(irows, o + 3)
            a1 = ld(jrows, o + 4) - ld(irows, o + 4)
            a2 = ld(jrows, o + 5) - ld(irows, o + 5)
            r2 = jnp.maximum(rx * rx + ry * ry + rz * rz, 1e-16)
            inv_s = _rsqrt_nr(r2)
            rs = r2 * inv_s
            q = rs * _INV_H
            b1 = q * (2.25 * q - 3.0)
            tq = 2.0 - q
            b2 = -0.75 * tq * tq
            dwdq = jnp.where(q < 1.0, b1,
                             jnp.where(q < 2.0, b2, jnp.float32(0.0)))
            kf = (_KC * vj) * (dwdq * inv_s)
            b0v = kf * rx
            b1v = kf * ry
            b2v = kf * rz
            st(8 * t + 0, a0 * b0v)
            st(8 * t + 1, a1 * b1v)
            st(8 * t + 2, a2 * b2v)
            st(8 * t + 3, a0 * b1v + a1 * b0v)
            st(8 * t + 4, a1 * b2v + a2 * b1v)
            st(8 * t + 5, a2 * b0v + a0 * b2v)


def _edge_kernel_body(tab_hbm, eij_hbm, out_hbm,
                      ibuf, jbuf, irows, jrows, contrib, accum,
                      sem_i, sem_g, sem_s,
                      *, n_pad, chunks_per_tile):
    c = lax.axis_index("c")
    s = lax.axis_index("s")
    npt = n_pad // _NS
    lanes = lax.iota(jnp.int32, _L)

    @pl.loop(0, _SLOTS * _CHUNK)
    def _(r):
        contrib[r // _CHUNK, r % _CHUNK, :] = jnp.zeros((_L,), jnp.float32)

    nbase = s * npt

    @pl.loop(0, (npt + _CHUNK - 1) // _CHUNK)
    def _(z):
        off = jnp.minimum(z * _CHUNK, npt - _CHUNK)
        pltpu.sync_copy(contrib.at[0], accum.at[pl.ds(nbase + off, _CHUNK)])

    plsc.subcore_barrier()

    w = c * _NS + s
    rowbase = w * chunks_per_tile

    @pl.loop(0, chunks_per_tile // _IBLK)
    def _(m):
        blk = rowbase + m * _IBLK
        pltpu.sync_copy(eij_hbm.at[0, pl.ds(blk, _IBLK)], ibuf)
        pltpu.sync_copy(eij_hbm.at[1, pl.ds(blk, _IBLK)], jbuf)

        gath = [None] * _IBLK
        scat = [None] * _IBLK

        def fire(k):
            sl = k % _SLOTS
            gath[k] = (
                pltpu.async_copy(tab_hbm.at[ibuf.at[k]], irows.at[sl],
                                 sem_g[sl]),
                pltpu.async_copy(tab_hbm.at[jbuf.at[k]], jrows.at[sl],
                                 sem_g[sl]),
            )

        for k in range(_SLOTS - 1):
            fire(k)
        for k in range(_IBLK):
            sl = k % _SLOTS
            if k + _SLOTS - 1 < _IBLK:
                fire(k + _SLOTS - 1)
            d1, d2 = gath[k]
            d1.wait()
            d2.wait()
            if k >= _SLOTS:
                scat[k - _SLOTS].wait()
            _compute_chunk(irows.at[sl], jrows.at[sl], contrib.at[sl], lanes)
            scat[k] = pltpu.async_copy(contrib.at[sl], accum.at[ibuf.at[k]],
                                       sem_s[sl], add=True)
        for k in range(_IBLK - _SLOTS, _IBLK):
            scat[k].wait()

    plsc.subcore_barrier()
    pltpu.sync_copy(accum.at[pl.ds(nbase, npt)],
                    out_hbm.at[c, pl.ds(nbase, npt)])


def _sc_edge_pass(table, eij, n_pad, chunks_per_tile):
    mesh = plsc.VectorSubcoreMesh(core_axis_name="c", subcore_axis_name="s",
                                  num_cores=_NC, num_subcores=_NS)
    body = functools.partial(_edge_kernel_body, n_pad=n_pad,
                             chunks_per_tile=chunks_per_tile)
    cp = pltpu.CompilerParams()
    for fld, val in (("needs_layout_passes", False),
                     ("use_tc_tiling_on_sc", False)):
        if fld in pltpu.CompilerParams.__dataclass_fields__:
            cp = dataclasses.replace(cp, **{fld: val})

    def wrapped(tab_hbm, eij_hbm, out_hbm, ibuf, jbuf, irows, jrows,
                contrib, accum, sem_i, sg0, sg1, sg2, sg3, ss0, ss1, ss2,
                ss3):
        _edge_kernel_body(tab_hbm, eij_hbm, out_hbm, ibuf, jbuf,
                          irows, jrows, contrib, accum, sem_i,
                          [sg0, sg1, sg2, sg3], [ss0, ss1, ss2, ss3],
                          n_pad=n_pad, chunks_per_tile=chunks_per_tile)

    return pl.kernel(
        wrapped,
        out_type=jax.ShapeDtypeStruct((_NC, n_pad, 16), jnp.float32),
        mesh=mesh,
        compiler_params=cp,
        scratch_types=[
            pltpu.VMEM((_IBLK, _CHUNK), jnp.int32),
            pltpu.VMEM((_IBLK, _CHUNK), jnp.int32),
            pltpu.VMEM((_SLOTS, _CHUNK, 16), jnp.float32),
            pltpu.VMEM((_SLOTS, _CHUNK, 16), jnp.float32),
            pltpu.VMEM((_SLOTS, _CHUNK, 16), jnp.float32),
            pltpu.VMEM_SHARED((n_pad, 16), jnp.float32),
        ] + [pltpu.SemaphoreType.DMA] * 9,
    )(table, eij)


def _node_kernel_body(p0, p1, e1m, e2m, en, v8, vc, uc, bvc, btc, out):
    x = p0[:, :] + p1[:, :]
    trm = jnp.dot(x, e1m[:, :], preferred_element_type=jnp.float32)
    qm = jnp.dot(x * x, e2m[:, :], preferred_element_type=jnp.float32)
    pi_int = jnp.float32(0.0)
    pi_ext = jnp.float32(0.0)
    for t in (0, 1):
        e_mod = en[2 * t]
        nu = en[2 * t + 1]
        one_m2nu = 1.0 - 2.0 * nu
        lam = e_mod / ((1.0 + nu) * one_m2nu)
        c1 = (nu * lam) * v8[:, :]
        c2 = (one_m2nu * lam) * v8[:, :]
        tr = trm[:, 8 * t:8 * t + 8]
        q = qm[:, 8 * t:8 * t + 8]
        pi_sum = jnp.sum(c1 * tr * tr + c2 * q)
        pi_int = pi_int + 0.5 * jnp.maximum(pi_sum, 0.0)
        dot = (bvc[3 * t] * uc[3 * t] + bvc[3 * t + 1] * uc[3 * t + 1]
               + bvc[3 * t + 2] * uc[3 * t + 2])
        masked = jnp.where(btc[t] == 2.0, dot, jnp.float32(0.0))
        pi_ext = pi_ext - jnp.sum(masked * vc[:, :])
    out[0, 0] = 0.5 * pi_int + 0.5 * pi_ext


def _node_pass(p0, p1, e1m, e2m, en, v8, vc, uc, bvc, btc):
    return pl.pallas_call(
        _node_kernel_body,
        out_shape=jax.ShapeDtypeStruct((1, 1), jnp.float32),
        out_specs=pl.BlockSpec(memory_space=pltpu.SMEM),
    )(p0, p1, e1m, e2m, en, v8, vc, uc, bvc, btc)


def kernel(displacement, coords, properties, volume, bc_values, domain_id,
           bc_type, edge_index):
    n = displacement.shape[0]
    e = edge_index.shape[1]
    f32 = jnp.float32

    table = jnp.concatenate([
        coords[:, 0, :], displacement[:, 0, :],
        coords[:, 1, :], displacement[:, 1, :],
        volume.astype(f32), jnp.zeros((n, 3), f32),
    ], axis=1)

    grain = _NW * _CHUNK * _IBLK
    e_pad = -(-e // grain) * grain
    chunks_per_tile = e_pad // (_NW * _CHUNK)
    eij = edge_index.astype(jnp.int32)
    if e_pad != e:
        zpad = jnp.arange(e_pad - e, dtype=jnp.int32) % jnp.int32(n)
        eij = jnp.concatenate([eij, jnp.broadcast_to(zpad, (2, e_pad - e))],
                              axis=1)
    eij = eij.reshape(2, e_pad // _CHUNK, _CHUNK)

    n_pad = -(-n // (_NS * 8)) * (_NS * 8)
    if n_pad != n:
        table = jnp.concatenate([table, jnp.zeros((n_pad - n, 16), f32)])

    partial = _sc_edge_pass(table, eij, n_pad, chunks_per_tile)

    rows = -(-n // 128)
    np2 = rows * 128

    def colpack(a2d, cols):
        x = jnp.stack([a2d[:n, k] for k in cols], 0)
        x = jnp.pad(x, ((0, 0), (0, np2 - n)))
        return x.reshape(len(cols), rows, 128)

    def colpack8(a2d, cols):
        x = jnp.stack([a2d[:n, k] for k in cols], 0)
        x = jnp.pad(x, ((0, 0), (0, n_pad - n)))
        return x.reshape(len(cols), n_pad // 8, 8)

    p0 = partial[0].reshape(n_pad // 8, 128)
    p1 = partial[1].reshape(n_pad // 8, 128)

    e1m = np.zeros((128, 16), np.float32)
    e2m = np.zeros((128, 16), np.float32)
    wq = [1.0, 1.0, 1.0, 0.5, 0.5, 0.5]
    for m in range(8):
        for t in range(2):
            for f in range(6):
                if f < 3:
                    e1m[16 * m + 8 * t + f, 8 * t + m] = 1.0
                e2m[16 * m + 8 * t + f, 8 * t + m] = wq[f]
    e1m = jnp.asarray(e1m)
    e2m = jnp.asarray(e2m)

    en = colpack8(properties.reshape(n, 4), [0, 1, 2, 3])
    v8 = colpack8(volume.astype(f32), [0])[0]
    vc = colpack(volume.astype(f32), [0])[0]
    uc = colpack(displacement.reshape(n, 6), list(range(6)))
    bvc = colpack(bc_values.reshape(n, 6), list(range(6)))
    btc = colpack(bc_type.astype(f32), [0, 1])

    res = _node_pass(p0, p1, e1m, e2m, en, v8, vc, uc, bvc, btc)
    return res[0, 0]


# speedup vs baseline: 519.3036x; 1.1102x over previous
import dataclasses
import functools
import math

import jax
import jax.numpy as jnp
from jax import lax
from jax.experimental import pallas as pl
from jax.experimental.pallas import tpu as pltpu
from jax.experimental.pallas import tpu_sc as plsc

H_SMOOTH = 2.0
_SIGMA3 = 1.0 / math.pi
_KC = -_SIGMA3 / (H_SMOOTH ** 4)
_INV_H = 1.0 / H_SMOOTH

_NC = 2
_NS = 16
_NW = _NC * _NS
_L = 16
_CHUNK = 128
_IBLK = 16
_SLOTS = 4


def _rsqrt_nr(x):
    i = plsc.bitcast(x, jnp.int32)
    i = jnp.int32(0x5F3759DF) - lax.shift_right_logical(i, 1)
    y = plsc.bitcast(i, jnp.float32)
    xh = 0.5 * x
    for _ in range(3):
        y = y * (1.5 - xh * y * y)
    return y


def _compute_chunk(irows, jrows, contrib, lanes):
    @pl.loop(0, _CHUNK // _L, unroll=2)
    def _(g):
        rbase = g * _L + lanes

        def ld(ref, col):
            cidx = jnp.full((_L,), col, jnp.int32)
            return plsc.load_gather(ref, [rbase, cidx])

        def st(col, val):
            cidx = jnp.full((_L,), col, jnp.int32)
            plsc.store_scatter(contrib, [rbase, cidx], val)

        vj = ld(jrows, 12)
        for t in (0, 1):
            o = 6 * t
            rx = ld(jrows, o + 0) - ld(irows, o + 0)
            ry = ld(jrows, o + 1) - ld(irows, o + 1)
            rz = ld(jrows, o + 2) - ld(irows, o + 2)
            a0 = ld(jrows, o + 3) - ld(irows, o + 3)
            a1 = ld(jrows, o + 4) - ld(irows, o + 4)
            a2 = ld(jrows, o + 5) - ld(irows, o + 5)
            r2 = jnp.maximum(rx * rx + ry * ry + rz * rz, 1e-16)
            inv_s = _rsqrt_nr(r2)
            rs = r2 * inv_s
            q = rs * _INV_H
            b1 = q * (2.25 * q - 3.0)
            tq = 2.0 - q
            b2 = -0.75 * tq * tq
            dwdq = jnp.where(q < 1.0, b1,
                             jnp.where(q < 2.0, b2, jnp.float32(0.0)))
            kf = (_KC * vj) * (dwdq * inv_s)
            b0v = kf * rx
            b1v = kf * ry
            b2v = kf * rz
            st(8 * t + 0, a0 * b0v)
            st(8 * t + 1, a1 * b1v)
            st(8 * t + 2, a2 * b2v)
            st(8 * t + 3, a0 * b1v + a1 * b0v)
            st(8 * t + 4, a1 * b2v + a2 * b1v)
            st(8 * t + 5, a2 * b0v + a0 * b2v)


def _edge_kernel_body(tab_hbm, eij_hbm, out_hbm,
                      ibuf, jbuf, irows, jrows, contrib, accum,
                      sem_i, sem_g, sem_s,
                      *, n_pad, chunks_per_tile):
    c = lax.axis_index("c")
    s = lax.axis_index("s")
    npt = n_pad // _NS
    lanes = lax.iota(jnp.int32, _L)

    @pl.loop(0, _SLOTS * _CHUNK)
    def _(r):
        contrib[r // _CHUNK, r % _CHUNK, :] = jnp.zeros((_L,), jnp.float32)

    nbase = s * npt

    @pl.loop(0, (npt + _CHUNK - 1) // _CHUNK)
    def _(z):
        off = jnp.minimum(z * _CHUNK, npt - _CHUNK)
        pltpu.sync_copy(contrib.at[0], accum.at[pl.ds(nbase + off, _CHUNK)])

    plsc.subcore_barrier()

    w = c * _NS + s
    rowbase = w * chunks_per_tile

    @pl.loop(0, chunks_per_tile // _IBLK)
    def _(m):
        blk = rowbase + m * _IBLK
        pltpu.sync_copy(eij_hbm.at[0, pl.ds(blk, _IBLK)], ibuf)
        pltpu.sync_copy(eij_hbm.at[1, pl.ds(blk, _IBLK)], jbuf)

        gath = [None] * _IBLK
        scat = [None] * _IBLK

        def fire(k):
            sl = k % _SLOTS
            gath[k] = (
                pltpu.async_copy(tab_hbm.at[ibuf.at[k]], irows.at[sl],
                                 sem_g[sl]),
                pltpu.async_copy(tab_hbm.at[jbuf.at[k]], jrows.at[sl],
                                 sem_g[sl]),
            )

        for k in range(_SLOTS - 1):
            fire(k)
        for k in range(_IBLK):
            sl = k % _SLOTS
            if k + _SLOTS - 1 < _IBLK:
                fire(k + _SLOTS - 1)
            d1, d2 = gath[k]
            d1.wait()
            d2.wait()
            if k >= _SLOTS:
                scat[k - _SLOTS].wait()
            _compute_chunk(irows.at[sl], jrows.at[sl], contrib.at[sl], lanes)
            scat[k] = pltpu.async_copy(contrib.at[sl], accum.at[ibuf.at[k]],
                                       sem_s[sl], add=True)
        for k in range(_IBLK - _SLOTS, _IBLK):
            scat[k].wait()

    plsc.subcore_barrier()
    pltpu.sync_copy(accum.at[pl.ds(nbase, npt)],
                    out_hbm.at[c, pl.ds(nbase, npt)])


def _sc_edge_pass(table, eij, n_pad, chunks_per_tile):
    mesh = plsc.VectorSubcoreMesh(core_axis_name="c", subcore_axis_name="s",
                                  num_cores=_NC, num_subcores=_NS)
    body = functools.partial(_edge_kernel_body, n_pad=n_pad,
                             chunks_per_tile=chunks_per_tile)
    cp = pltpu.CompilerParams()
    for fld, val in (("needs_layout_passes", False),
                     ("use_tc_tiling_on_sc", False)):
        if fld in pltpu.CompilerParams.__dataclass_fields__:
            cp = dataclasses.replace(cp, **{fld: val})

    def wrapped(tab_hbm, eij_hbm, out_hbm, ibuf, jbuf, irows, jrows,
                contrib, accum, sem_i, sg0, sg1, sg2, sg3, ss0, ss1, ss2,
                ss3):
        _edge_kernel_body(tab_hbm, eij_hbm, out_hbm, ibuf, jbuf,
                          irows, jrows, contrib, accum, sem_i,
                          [sg0, sg1, sg2, sg3], [ss0, ss1, ss2, ss3],
                          n_pad=n_pad, chunks_per_tile=chunks_per_tile)

    return pl.kernel(
        wrapped,
        out_type=jax.ShapeDtypeStruct((_NC, n_pad, 16), jnp.float32),
        mesh=mesh,
        compiler_params=cp,
        scratch_types=[
            pltpu.VMEM((_IBLK, _CHUNK), jnp.int32),
            pltpu.VMEM((_IBLK, _CHUNK), jnp.int32),
            pltpu.VMEM((_SLOTS, _CHUNK, 16), jnp.float32),
            pltpu.VMEM((_SLOTS, _CHUNK, 16), jnp.float32),
            pltpu.VMEM((_SLOTS, _CHUNK, 16), jnp.float32),
            pltpu.VMEM_SHARED((n_pad, 16), jnp.float32),
        ] + [pltpu.SemaphoreType.DMA] * 9,
    )(table, eij)


def _node_sc_body(part_hbm, t2_hbm, out_hbm, pa, pb, t2v, obuf, sem,
                  *, n_pad):
    c = lax.axis_index("c")
    s = lax.axis_index("s")
    w = c * _NS + s
    npw = n_pad // _NW
    base = w * npw
    d1 = pltpu.async_copy(part_hbm.at[0, pl.ds(base, npw)], pa, sem)
    d2 = pltpu.async_copy(part_hbm.at[1, pl.ds(base, npw)], pb, sem)
    d3 = pltpu.async_copy(t2_hbm.at[pl.ds(base, npw)], t2v, sem)
    d1.wait()
    d2.wait()
    d3.wait()
    lanes = lax.iota(jnp.int32, _L)
    zero = jnp.zeros((_L,), jnp.float32)

    @pl.loop(0, npw // _L, init_carry=(zero, zero, zero, zero), unroll=2)
    def carry(g, acc):
        it0, it1, et0, et1 = acc
        rb = g * _L + lanes

        def ld(ref, col):
            cidx = jnp.full((_L,), col, jnp.int32)
            return plsc.load_gather(ref, [rb, cidx])

        vol = ld(t2v, 4)
        its = []
        ets = []
        for t in (0, 1):
            sv = [ld(pa, 8 * t + k) + ld(pb, 8 * t + k) for k in range(6)]
            e_mod = ld(t2v, 2 * t)
            nu = ld(t2v, 2 * t + 1)
            one_m2nu = 1.0 - 2.0 * nu
            lam = e_mod / ((1.0 + nu) * one_m2nu)
            tr = sv[0] + sv[1] + sv[2]
            q = (sv[0] * sv[0] + sv[1] * sv[1] + sv[2] * sv[2]
                 + 0.5 * (sv[3] * sv[3] + sv[4] * sv[4] + sv[5] * sv[5]))
            its.append(vol * ((nu * lam) * tr * tr + (one_m2nu * lam) * q))
            dot = (ld(t2v, 5 + 3 * t) * ld(t2v, 11 + 3 * t)
                   + ld(t2v, 6 + 3 * t) * ld(t2v, 12 + 3 * t)
                   + ld(t2v, 7 + 3 * t) * ld(t2v, 13 + 3 * t))
            msk = jnp.where(ld(t2v, 17 + t) == 2.0, dot, jnp.float32(0.0))
            ets.append(msk * vol)
        return (it0 + its[0], it1 + its[1], et0 + ets[0], et1 + ets[1])

    obuf[0, :] = carry[0]
    obuf[1, :] = carry[1]
    obuf[2, :] = carry[2]
    obuf[3, :] = carry[3]
    pltpu.sync_copy(obuf, out_hbm.at[c, s])


def _node_sc_pass(partial, t2, n_pad):
    mesh = plsc.VectorSubcoreMesh(core_axis_name="c", subcore_axis_name="s",
                                  num_cores=_NC, num_subcores=_NS)
    cp = pltpu.CompilerParams()
    for fld, val in (("needs_layout_passes", False),
                     ("use_tc_tiling_on_sc", False)):
        if fld in pltpu.CompilerParams.__dataclass_fields__:
            cp = dataclasses.replace(cp, **{fld: val})
    npw = n_pad // _NW
    return pl.kernel(
        functools.partial(_node_sc_body, n_pad=n_pad),
        out_type=jax.ShapeDtypeStruct((_NC, _NS, 4, _L), jnp.float32),
        mesh=mesh,
        compiler_params=cp,
        scratch_types=[
            pltpu.VMEM((npw, 16), jnp.float32),
            pltpu.VMEM((npw, 16), jnp.float32),
            pltpu.VMEM((npw, 24), jnp.float32),
            pltpu.VMEM((4, _L), jnp.float32),
            pltpu.SemaphoreType.DMA,
        ],
    )(partial, t2)


def _combine_body(x, out):
    it0 = jnp.sum(x[:, :, 0, :])
    it1 = jnp.sum(x[:, :, 1, :])
    et0 = jnp.sum(x[:, :, 2, :])
    et1 = jnp.sum(x[:, :, 3, :])
    pi_int = 0.5 * (0.5 * jnp.maximum(it0, 0.0)
                    + 0.5 * jnp.maximum(it1, 0.0))
    pi_ext = 0.5 * (-et0 - et1)
    out[0, 0] = pi_int + pi_ext


def _combine_pass(x):
    return pl.pallas_call(
        _combine_body,
        out_shape=jax.ShapeDtypeStruct((1, 1), jnp.float32),
        out_specs=pl.BlockSpec(memory_space=pltpu.SMEM),
    )(x)


def kernel(displacement, coords, properties, volume, bc_values, domain_id,
           bc_type, edge_index):
    n = displacement.shape[0]
    e = edge_index.shape[1]
    f32 = jnp.float32

    table = jnp.concatenate([
        coords[:, 0, :], displacement[:, 0, :],
        coords[:, 1, :], displacement[:, 1, :],
        volume.astype(f32), jnp.zeros((n, 3), f32),
    ], axis=1)

    grain = _NW * _CHUNK * _IBLK
    e_pad = -(-e // grain) * grain
    chunks_per_tile = e_pad // (_NW * _CHUNK)
    eij = edge_index.astype(jnp.int32)
    if e_pad != e:
        zpad = jnp.arange(e_pad - e, dtype=jnp.int32) % jnp.int32(n)
        eij = jnp.concatenate([eij, jnp.broadcast_to(zpad, (2, e_pad - e))],
                              axis=1)
    eij = eij.reshape(2, e_pad // _CHUNK, _CHUNK)

    n_pad = -(-n // (_NW * 8)) * (_NW * 8)
    if n_pad != n:
        table = jnp.concatenate([table, jnp.zeros((n_pad - n, 16), f32)])

    t2 = jnp.concatenate([
        properties.reshape(n, 4),
        volume.astype(f32),
        displacement.reshape(n, 6),
        bc_values.reshape(n, 6),
        bc_type.astype(f32),
        jnp.zeros((n, 5), f32),
    ], axis=1)
    if n_pad != n:
        t2 = jnp.concatenate([t2, jnp.zeros((n_pad - n, 24), f32)])

    partial = _sc_edge_pass(table, eij, n_pad, chunks_per_tile)
    sums = _node_sc_pass(partial, t2, n_pad)
    res = _combine_pass(sums)
    return res[0, 0]


# speedup vs baseline: 533.9479x; 1.0282x over previous
import dataclasses
import functools
import math

import jax
import jax.numpy as jnp
from jax import lax
from jax.experimental import pallas as pl
from jax.experimental.pallas import tpu as pltpu
from jax.experimental.pallas import tpu_sc as plsc

H_SMOOTH = 2.0
_SIGMA3 = 1.0 / math.pi
_KC = -_SIGMA3 / (H_SMOOTH ** 4)
_INV_H = 1.0 / H_SMOOTH

_NC = 2
_NS = 16
_NW = _NC * _NS
_L = 16
_CHUNK = 128
_IBLK = 16
_SLOTS = 6


def _rsqrt_nr(x):
    i = plsc.bitcast(x, jnp.int32)
    i = jnp.int32(0x5F3759DF) - lax.shift_right_logical(i, 1)
    y = plsc.bitcast(i, jnp.float32)
    xh = 0.5 * x
    for _ in range(2):
        y = y * (1.5 - xh * y * y)
    return y


def _compute_chunk(irows, jrows, contrib, lanes):
    @pl.loop(0, _CHUNK // _L, unroll=2)
    def _(g):
        rbase = g * _L + lanes

        def ld(ref, col):
            cidx = jnp.full((_L,), col, jnp.int32)
            return plsc.load_gather(ref, [rbase, cidx])

        def st(col, val):
            cidx = jnp.full((_L,), col, jnp.int32)
            plsc.store_scatter(contrib, [rbase, cidx], val)

        vj = ld(jrows, 12)
        for t in (0, 1):
            o = 6 * t
            rx = ld(jrows, o + 0) - ld(irows, o + 0)
            ry = ld(jrows, o + 1) - ld(irows, o + 1)
            rz = ld(jrows, o + 2) - ld(irows, o + 2)
            a0 = ld(jrows, o + 3) - ld(irows, o + 3)
            a1 = ld(jrows, o + 4) - ld(irows, o + 4)
            a2 = ld(jrows, o + 5) - ld(irows, o + 5)
            r2 = jnp.maximum(rx * rx + ry * ry + rz * rz, 1e-16)
            inv_s = _rsqrt_nr(r2)
            rs = r2 * inv_s
            q = rs * _INV_H
            b1 = q * (2.25 * q - 3.0)
            tq = 2.0 - q
            b2 = -0.75 * tq * tq
            dwdq = jnp.where(q < 1.0, b1,
                             jnp.where(q < 2.0, b2, jnp.float32(0.0)))
            kf = (_KC * vj) * (dwdq * inv_s)
            b0v = kf * rx
            b1v = kf * ry
            b2v = kf * rz
            st(8 * t + 0, a0 * b0v)
            st(8 * t + 1, a1 * b1v)
            st(8 * t + 2, a2 * b2v)
            st(8 * t + 3, a0 * b1v + a1 * b0v)
            st(8 * t + 4, a1 * b2v + a2 * b1v)
            st(8 * t + 5, a2 * b0v + a0 * b2v)


def _edge_kernel_body(tab_hbm, eij_hbm, out_hbm,
                      ibuf, jbuf, irows, jrows, contrib, accum,
                      sem_i, sem_g, sem_s,
                      *, n_pad, chunks_per_tile):
    c = lax.axis_index("c")
    s = lax.axis_index("s")
    npt = n_pad // _NS
    lanes = lax.iota(jnp.int32, _L)

    @pl.loop(0, _SLOTS * _CHUNK)
    def _(r):
        contrib[r // _CHUNK, r % _CHUNK, :] = jnp.zeros((_L,), jnp.float32)

    nbase = s * npt

    @pl.loop(0, (npt + _CHUNK - 1) // _CHUNK)
    def _(z):
        off = jnp.minimum(z * _CHUNK, npt - _CHUNK)
        pltpu.sync_copy(contrib.at[0], accum.at[pl.ds(nbase + off, _CHUNK)])

    plsc.subcore_barrier()

    w = c * _NS + s
    rowbase = w * chunks_per_tile

    @pl.loop(0, chunks_per_tile // _IBLK)
    def _(m):
        blk = rowbase + m * _IBLK
        pltpu.sync_copy(eij_hbm.at[0, pl.ds(blk, _IBLK)], ibuf)
        pltpu.sync_copy(eij_hbm.at[1, pl.ds(blk, _IBLK)], jbuf)

        gath = [None] * _IBLK
        scat = [None] * _IBLK

        def fire(k):
            sl = k % _SLOTS
            gath[k] = (
                pltpu.async_copy(tab_hbm.at[ibuf.at[k]], irows.at[sl],
                                 sem_g[sl]),
                pltpu.async_copy(tab_hbm.at[jbuf.at[k]], jrows.at[sl],
                                 sem_g[sl]),
            )

        for k in range(_SLOTS - 1):
            fire(k)
        for k in range(_IBLK):
            sl = k % _SLOTS
            if k + _SLOTS - 1 < _IBLK:
                fire(k + _SLOTS - 1)
            d1, d2 = gath[k]
            d1.wait()
            d2.wait()
            if k >= _SLOTS:
                scat[k - _SLOTS].wait()
            _compute_chunk(irows.at[sl], jrows.at[sl], contrib.at[sl], lanes)
            scat[k] = pltpu.async_copy(contrib.at[sl], accum.at[ibuf.at[k]],
                                       sem_s[sl], add=True)
        for k in range(_IBLK - _SLOTS, _IBLK):
            scat[k].wait()

    plsc.subcore_barrier()
    pltpu.sync_copy(accum.at[pl.ds(nbase, npt)],
                    out_hbm.at[c, pl.ds(nbase, npt)])


def _sc_edge_pass(table, eij, n_pad, chunks_per_tile):
    mesh = plsc.VectorSubcoreMesh(core_axis_name="c", subcore_axis_name="s",
                                  num_cores=_NC, num_subcores=_NS)
    body = functools.partial(_edge_kernel_body, n_pad=n_pad,
                             chunks_per_tile=chunks_per_tile)
    cp = pltpu.CompilerParams()
    for fld, val in (("needs_layout_passes", False),
                     ("use_tc_tiling_on_sc", False)):
        if fld in pltpu.CompilerParams.__dataclass_fields__:
            cp = dataclasses.replace(cp, **{fld: val})

    def wrapped(tab_hbm, eij_hbm, out_hbm, ibuf, jbuf, irows, jrows,
                contrib, accum, sem_i, *sems):
        _edge_kernel_body(tab_hbm, eij_hbm, out_hbm, ibuf, jbuf,
                          irows, jrows, contrib, accum, sem_i,
                          list(sems[:_SLOTS]), list(sems[_SLOTS:]),
                          n_pad=n_pad, chunks_per_tile=chunks_per_tile)

    return pl.kernel(
        wrapped,
        out_type=jax.ShapeDtypeStruct((_NC, n_pad, 16), jnp.float32),
        mesh=mesh,
        compiler_params=cp,
        scratch_types=[
            pltpu.VMEM((_IBLK, _CHUNK), jnp.int32),
            pltpu.VMEM((_IBLK, _CHUNK), jnp.int32),
            pltpu.VMEM((_SLOTS, _CHUNK, 16), jnp.float32),
            pltpu.VMEM((_SLOTS, _CHUNK, 16), jnp.float32),
            pltpu.VMEM((_SLOTS, _CHUNK, 16), jnp.float32),
            pltpu.VMEM_SHARED((n_pad, 16), jnp.float32),
        ] + [pltpu.SemaphoreType.DMA] * (1 + 2 * _SLOTS),
    )(table, eij)


def _node_sc_body(part_hbm, t2_hbm, out_hbm, pa, pb, t2v, obuf, sem,
                  *, n_pad):
    c = lax.axis_index("c")
    s = lax.axis_index("s")
    w = c * _NS + s
    npw = n_pad // _NW
    base = w * npw
    d1 = pltpu.async_copy(part_hbm.at[0, pl.ds(base, npw)], pa, sem)
    d2 = pltpu.async_copy(part_hbm.at[1, pl.ds(base, npw)], pb, sem)
    d3 = pltpu.async_copy(t2_hbm.at[pl.ds(base, npw)], t2v, sem)
    d1.wait()
    d2.wait()
    d3.wait()
    lanes = lax.iota(jnp.int32, _L)
    zero = jnp.zeros((_L,), jnp.float32)

    @pl.loop(0, npw // _L, init_carry=(zero, zero, zero, zero), unroll=2)
    def carry(g, acc):
        it0, it1, et0, et1 = acc
        rb = g * _L + lanes

        def ld(ref, col):
            cidx = jnp.full((_L,), col, jnp.int32)
            return plsc.load_gather(ref, [rb, cidx])

        vol = ld(t2v, 4)
        its = []
        ets = []
        for t in (0, 1):
            sv = [ld(pa, 8 * t + k) + ld(pb, 8 * t + k) for k in range(6)]
            e_mod = ld(t2v, 2 * t)
            nu = ld(t2v, 2 * t + 1)
            one_m2nu = 1.0 - 2.0 * nu
            lam = e_mod / ((1.0 + nu) * one_m2nu)
            tr = sv[0] + sv[1] + sv[2]
            q = (sv[0] * sv[0] + sv[1] * sv[1] + sv[2] * sv[2]
                 + 0.5 * (sv[3] * sv[3] + sv[4] * sv[4] + sv[5] * sv[5]))
            its.append(vol * ((nu * lam) * tr * tr + (one_m2nu * lam) * q))
            dot = (ld(t2v, 5 + 3 * t) * ld(t2v, 11 + 3 * t)
                   + ld(t2v, 6 + 3 * t) * ld(t2v, 12 + 3 * t)
                   + ld(t2v, 7 + 3 * t) * ld(t2v, 13 + 3 * t))
            msk = jnp.where(ld(t2v, 17 + t) == 2.0, dot, jnp.float32(0.0))
            ets.append(msk * vol)
        return (it0 + its[0], it1 + its[1], et0 + ets[0], et1 + ets[1])

    obuf[0, :] = carry[0]
    obuf[1, :] = carry[1]
    obuf[2, :] = carry[2]
    obuf[3, :] = carry[3]
    pltpu.sync_copy(obuf, out_hbm.at[c, s])


def _node_sc_pass(partial, t2, n_pad):
    mesh = plsc.VectorSubcoreMesh(core_axis_name="c", subcore_axis_name="s",
                                  num_cores=_NC, num_subcores=_NS)
    cp = pltpu.CompilerParams()
    for fld, val in (("needs_layout_passes", False),
                     ("use_tc_tiling_on_sc", False)):
        if fld in pltpu.CompilerParams.__dataclass_fields__:
            cp = dataclasses.replace(cp, **{fld: val})
    npw = n_pad // _NW
    return pl.kernel(
        functools.partial(_node_sc_body, n_pad=n_pad),
        out_type=jax.ShapeDtypeStruct((_NC, _NS, 4, _L), jnp.float32),
        mesh=mesh,
        compiler_params=cp,
        scratch_types=[
            pltpu.VMEM((npw, 16), jnp.float32),
            pltpu.VMEM((npw, 16), jnp.float32),
            pltpu.VMEM((npw, 24), jnp.float32),
            pltpu.VMEM((4, _L), jnp.float32),
            pltpu.SemaphoreType.DMA,
        ],
    )(partial, t2)


def _combine_body(x, out):
    it0 = jnp.sum(x[:, :, 0, :])
    it1 = jnp.sum(x[:, :, 1, :])
    et0 = jnp.sum(x[:, :, 2, :])
    et1 = jnp.sum(x[:, :, 3, :])
    pi_int = 0.5 * (0.5 * jnp.maximum(it0, 0.0)
                    + 0.5 * jnp.maximum(it1, 0.0))
    pi_ext = 0.5 * (-et0 - et1)
    out[0, 0] = pi_int + pi_ext


def _combine_pass(x):
    return pl.pallas_call(
        _combine_body,
        out_shape=jax.ShapeDtypeStruct((1, 1), jnp.float32),
        out_specs=pl.BlockSpec(memory_space=pltpu.SMEM),
    )(x)


def kernel(displacement, coords, properties, volume, bc_values, domain_id,
           bc_type, edge_index):
    n = displacement.shape[0]
    e = edge_index.shape[1]
    f32 = jnp.float32

    table = jnp.concatenate([
        coords[:, 0, :], displacement[:, 0, :],
        coords[:, 1, :], displacement[:, 1, :],
        volume.astype(f32), jnp.zeros((n, 3), f32),
    ], axis=1)

    grain = _NW * _CHUNK * _IBLK
    e_pad = -(-e // grain) * grain
    chunks_per_tile = e_pad // (_NW * _CHUNK)
    eij = edge_index.astype(jnp.int32)
    if e_pad != e:
        zpad = jnp.arange(e_pad - e, dtype=jnp.int32) % jnp.int32(n)
        eij = jnp.concatenate([eij, jnp.broadcast_to(zpad, (2, e_pad - e))],
                              axis=1)
    eij = eij.reshape(2, e_pad // _CHUNK, _CHUNK)

    n_pad = -(-n // (_NW * 8)) * (_NW * 8)
    if n_pad != n:
        table = jnp.concatenate([table, jnp.zeros((n_pad - n, 16), f32)])

    t2 = jnp.concatenate([
        properties.reshape(n, 4),
        volume.astype(f32),
        displacement.reshape(n, 6),
        bc_values.reshape(n, 6),
        bc_type.astype(f32),
        jnp.zeros((n, 5), f32),
    ], axis=1)
    if n_pad != n:
        t2 = jnp.concatenate([t2, jnp.zeros((n_pad - n, 24), f32)])

    partial = _sc_edge_pass(table, eij, n_pad, chunks_per_tile)
    sums = _node_sc_pass(partial, t2, n_pad)
    res = _combine_pass(sums)
    return res[0, 0]


# speedup vs baseline: 538.4709x; 1.0085x over previous
import dataclasses
import functools
import math

import jax
import jax.numpy as jnp
from jax import lax
from jax.experimental import pallas as pl
from jax.experimental.pallas import tpu as pltpu
from jax.experimental.pallas import tpu_sc as plsc

H_SMOOTH = 2.0
_SIGMA3 = 1.0 / math.pi
_KC = -_SIGMA3 / (H_SMOOTH ** 4)
_INV_H = 1.0 / H_SMOOTH

_NC = 2
_NS = 16
_NW = _NC * _NS
_L = 16
_CHUNK = 128
_IBLK = 16
_SLOTS = 6


def _rsqrt_nr(x):
    i = plsc.bitcast(x, jnp.int32)
    i = jnp.int32(0x5F3759DF) - lax.shift_right_logical(i, 1)
    y = plsc.bitcast(i, jnp.float32)
    xh = 0.5 * x
    for _ in range(2):
        y = y * (1.5 - xh * y * y)
    return y


def _compute_chunk(irows, jrows, contrib, lanes):
    @pl.loop(0, _CHUNK // _L, unroll=2)
    def _(g):
        rbase = g * _L + lanes

        def ld(ref, col):
            cidx = jnp.full((_L,), col, jnp.int32)
            return plsc.load_gather(ref, [rbase, cidx])

        def st(col, val):
            cidx = jnp.full((_L,), col, jnp.int32)
            plsc.store_scatter(contrib, [rbase, cidx], val)

        vj = ld(jrows, 12)
        for t in (0, 1):
            o = 6 * t
            rx = ld(jrows, o + 0) - ld(irows, o + 0)
            ry = ld(jrows, o + 1) - ld(irows, o + 1)
            rz = ld(jrows, o + 2) - ld(irows, o + 2)
            a0 = ld(jrows, o + 3) - ld(irows, o + 3)
            a1 = ld(jrows, o + 4) - ld(irows, o + 4)
            a2 = ld(jrows, o + 5) - ld(irows, o + 5)
            r2 = jnp.maximum(rx * rx + ry * ry + rz * rz, 1e-16)
            inv_s = _rsqrt_nr(r2)
            rs = r2 * inv_s
            q = rs * _INV_H
            b1 = q * (2.25 * q - 3.0)
            tq = 2.0 - q
            b2 = -0.75 * tq * tq
            dwdq = jnp.where(q < 1.0, b1,
                             jnp.where(q < 2.0, b2, jnp.float32(0.0)))
            kf = (_KC * vj) * (dwdq * inv_s)
            b0v = kf * rx
            b1v = kf * ry
            b2v = kf * rz
            st(8 * t + 0, a0 * b0v)
            st(8 * t + 1, a1 * b1v)
            st(8 * t + 2, a2 * b2v)
            st(8 * t + 3, a0 * b1v + a1 * b0v)
            st(8 * t + 4, a1 * b2v + a2 * b1v)
            st(8 * t + 5, a2 * b0v + a0 * b2v)


def _edge_kernel_body(tab_hbm, eij_hbm, out_hbm,
                      ibuf, jbuf, irows, jrows, contrib, accum,
                      sem_i, sem_g, sem_s,
                      *, n_pad, total_chunks):
    c = lax.axis_index("c")
    s = lax.axis_index("s")
    npt = n_pad // _NS
    lanes = lax.iota(jnp.int32, _L)

    @pl.loop(0, _SLOTS * _CHUNK)
    def _(r):
        contrib[r // _CHUNK, r % _CHUNK, :] = jnp.zeros((_L,), jnp.float32)

    nbase = s * npt

    @pl.loop(0, (npt + _CHUNK - 1) // _CHUNK)
    def _(z):
        off = jnp.minimum(z * _CHUNK, npt - _CHUNK)
        pltpu.sync_copy(contrib.at[0], accum.at[pl.ds(nbase + off, _CHUNK)])

    plsc.subcore_barrier()

    w = c * _NS + s
    full_blocks = total_chunks // _IBLK
    rem = total_chunks % _IBLK
    blk_base = full_blocks // _NW
    blk_extra = full_blocks % _NW
    nblk = blk_base + jnp.where(w < blk_extra, 1, 0)
    row0 = (w * blk_base + jnp.minimum(w, blk_extra)) * _IBLK

    @pl.loop(0, nblk)
    def _(m):
        blk = row0 + m * _IBLK
        pltpu.sync_copy(eij_hbm.at[0, pl.ds(blk, _IBLK)], ibuf)
        pltpu.sync_copy(eij_hbm.at[1, pl.ds(blk, _IBLK)], jbuf)

        gath = [None] * _IBLK
        scat = [None] * _IBLK

        def fire(k):
            sl = k % _SLOTS
            gath[k] = (
                pltpu.async_copy(tab_hbm.at[ibuf.at[k]], irows.at[sl],
                                 sem_g[sl]),
                pltpu.async_copy(tab_hbm.at[jbuf.at[k]], jrows.at[sl],
                                 sem_g[sl]),
            )

        for k in range(_SLOTS - 1):
            fire(k)
        for k in range(_IBLK):
            sl = k % _SLOTS
            if k + _SLOTS - 1 < _IBLK:
                fire(k + _SLOTS - 1)
            d1, d2 = gath[k]
            d1.wait()
            d2.wait()
            if k >= _SLOTS:
                scat[k - _SLOTS].wait()
            _compute_chunk(irows.at[sl], jrows.at[sl], contrib.at[sl], lanes)
            scat[k] = pltpu.async_copy(contrib.at[sl], accum.at[ibuf.at[k]],
                                       sem_s[sl], add=True)
        for k in range(_IBLK - _SLOTS, _IBLK):
            scat[k].wait()

    if rem:
        @pl.when(w == _NW - 1)
        def _():
            for k in range(rem):
                row = full_blocks * _IBLK + k
                pltpu.sync_copy(eij_hbm.at[0, row], ibuf.at[0])
                pltpu.sync_copy(eij_hbm.at[1, row], jbuf.at[0])
                pltpu.sync_copy(tab_hbm.at[ibuf.at[0]], irows.at[0])
                pltpu.sync_copy(tab_hbm.at[jbuf.at[0]], jrows.at[0])
                _compute_chunk(irows.at[0], jrows.at[0], contrib.at[0],
                               lanes)
                pltpu.sync_copy(contrib.at[0], accum.at[ibuf.at[0]],
                                add=True)

    plsc.subcore_barrier()
    pltpu.sync_copy(accum.at[pl.ds(nbase, npt)],
                    out_hbm.at[c, pl.ds(nbase, npt)])


def _sc_edge_pass(table, eij, n_pad, total_chunks):
    mesh = plsc.VectorSubcoreMesh(core_axis_name="c", subcore_axis_name="s",
                                  num_cores=_NC, num_subcores=_NS)
    cp = pltpu.CompilerParams()
    for fld, val in (("needs_layout_passes", False),
                     ("use_tc_tiling_on_sc", False)):
        if fld in pltpu.CompilerParams.__dataclass_fields__:
            cp = dataclasses.replace(cp, **{fld: val})

    def wrapped(tab_hbm, eij_hbm, out_hbm, ibuf, jbuf, irows, jrows,
                contrib, accum, sem_i, *sems):
        _edge_kernel_body(tab_hbm, eij_hbm, out_hbm, ibuf, jbuf,
                          irows, jrows, contrib, accum, sem_i,
                          list(sems[:_SLOTS]), list(sems[_SLOTS:]),
                          n_pad=n_pad, total_chunks=total_chunks)

    return pl.kernel(
        wrapped,
        out_type=jax.ShapeDtypeStruct((_NC, n_pad, 16), jnp.float32),
        mesh=mesh,
        compiler_params=cp,
        scratch_types=[
            pltpu.VMEM((_IBLK, _CHUNK), jnp.int32),
            pltpu.VMEM((_IBLK, _CHUNK), jnp.int32),
            pltpu.VMEM((_SLOTS, _CHUNK, 16), jnp.float32),
            pltpu.VMEM((_SLOTS, _CHUNK, 16), jnp.float32),
            pltpu.VMEM((_SLOTS, _CHUNK, 16), jnp.float32),
            pltpu.VMEM_SHARED((n_pad, 16), jnp.float32),
        ] + [pltpu.SemaphoreType.DMA] * (1 + 2 * _SLOTS),
    )(table, eij)


def _node_sc_body(part_hbm, t2_hbm, out_hbm, pa, pb, t2v, obuf, sem,
                  *, n_pad):
    c = lax.axis_index("c")
    s = lax.axis_index("s")
    w = c * _NS + s
    npw = n_pad // _NW
    base = w * npw
    d1 = pltpu.async_copy(part_hbm.at[0, pl.ds(base, npw)], pa, sem)
    d2 = pltpu.async_copy(part_hbm.at[1, pl.ds(base, npw)], pb, sem)
    d3 = pltpu.async_copy(t2_hbm.at[pl.ds(base, npw)], t2v, sem)
    d1.wait()
    d2.wait()
    d3.wait()
    lanes = lax.iota(jnp.int32, _L)
    zero = jnp.zeros((_L,), jnp.float32)

    @pl.loop(0, npw // _L, init_carry=(zero, zero, zero, zero), unroll=2)
    def carry(g, acc):
        it0, it1, et0, et1 = acc
        rb = g * _L + lanes

        def ld(ref, col):
            cidx = jnp.full((_L,), col, jnp.int32)
            return plsc.load_gather(ref, [rb, cidx])

        vol = ld(t2v, 4)
        its = []
        ets = []
        for t in (0, 1):
            sv = [ld(pa, 8 * t + k) + ld(pb, 8 * t + k) for k in range(6)]
            e_mod = ld(t2v, 2 * t)
            nu = ld(t2v, 2 * t + 1)
            one_m2nu = 1.0 - 2.0 * nu
            lam = e_mod / ((1.0 + nu) * one_m2nu)
            tr = sv[0] + sv[1] + sv[2]
            q = (sv[0] * sv[0] + sv[1] * sv[1] + sv[2] * sv[2]
                 + 0.5 * (sv[3] * sv[3] + sv[4] * sv[4] + sv[5] * sv[5]))
            its.append(vol * ((nu * lam) * tr * tr + (one_m2nu * lam) * q))
            dot = (ld(t2v, 5 + 3 * t) * ld(t2v, 11 + 3 * t)
                   + ld(t2v, 6 + 3 * t) * ld(t2v, 12 + 3 * t)
                   + ld(t2v, 7 + 3 * t) * ld(t2v, 13 + 3 * t))
            msk = jnp.where(ld(t2v, 17 + t) == 2.0, dot, jnp.float32(0.0))
            ets.append(msk * vol)
        return (it0 + its[0], it1 + its[1], et0 + ets[0], et1 + ets[1])

    obuf[0, :] = carry[0]
    obuf[1, :] = carry[1]
    obuf[2, :] = carry[2]
    obuf[3, :] = carry[3]
    pltpu.sync_copy(obuf, out_hbm.at[c, s])


def _node_sc_pass(partial, t2, n_pad):
    mesh = plsc.VectorSubcoreMesh(core_axis_name="c", subcore_axis_name="s",
                                  num_cores=_NC, num_subcores=_NS)
    cp = pltpu.CompilerParams()
    for fld, val in (("needs_layout_passes", False),
                     ("use_tc_tiling_on_sc", False)):
        if fld in pltpu.CompilerParams.__dataclass_fields__:
            cp = dataclasses.replace(cp, **{fld: val})
    npw = n_pad // _NW
    return pl.kernel(
        functools.partial(_node_sc_body, n_pad=n_pad),
        out_type=jax.ShapeDtypeStruct((_NC, _NS, 4, _L), jnp.float32),
        mesh=mesh,
        compiler_params=cp,
        scratch_types=[
            pltpu.VMEM((npw, 16), jnp.float32),
            pltpu.VMEM((npw, 16), jnp.float32),
            pltpu.VMEM((npw, 24), jnp.float32),
            pltpu.VMEM((4, _L), jnp.float32),
            pltpu.SemaphoreType.DMA,
        ],
    )(partial, t2)


def _combine_body(x, out):
    it0 = jnp.sum(x[:, :, 0, :])
    it1 = jnp.sum(x[:, :, 1, :])
    et0 = jnp.sum(x[:, :, 2, :])
    et1 = jnp.sum(x[:, :, 3, :])
    pi_int = 0.5 * (0.5 * jnp.maximum(it0, 0.0)
                    + 0.5 * jnp.maximum(it1, 0.0))
    pi_ext = 0.5 * (-et0 - et1)
    out[0, 0] = pi_int + pi_ext


def _combine_pass(x):
    return pl.pallas_call(
        _combine_body,
        out_shape=jax.ShapeDtypeStruct((1, 1), jnp.float32),
        out_specs=pl.BlockSpec(memory_space=pltpu.SMEM),
    )(x)


def kernel(displacement, coords, properties, volume, bc_values, domain_id,
           bc_type, edge_index):
    n = displacement.shape[0]
    e = edge_index.shape[1]
    f32 = jnp.float32

    table = jnp.concatenate([
        coords[:, 0, :], displacement[:, 0, :],
        coords[:, 1, :], displacement[:, 1, :],
        volume.astype(f32), jnp.zeros((n, 3), f32),
    ], axis=1)

    e_pad = -(-e // _CHUNK) * _CHUNK
    eij = edge_index.astype(jnp.int32)
    if e_pad != e:
        zpad = jnp.arange(e_pad - e, dtype=jnp.int32) % jnp.int32(n)
        eij = jnp.concatenate([eij, jnp.broadcast_to(zpad, (2, e_pad - e))],
                              axis=1)
    total_chunks = e_pad // _CHUNK
    eij = eij.reshape(2, total_chunks, _CHUNK)

    n_pad = -(-n // (_NW * 8)) * (_NW * 8)
    if n_pad != n:
        table = jnp.concatenate([table, jnp.zeros((n_pad - n, 16), f32)])

    t2 = jnp.concatenate([
        properties.reshape(n, 4),
        volume.astype(f32),
        displacement.reshape(n, 6),
        bc_values.reshape(n, 6),
        bc_type.astype(f32),
        jnp.zeros((n, 5), f32),
    ], axis=1)
    if n_pad != n:
        t2 = jnp.concatenate([t2, jnp.zeros((n_pad - n, 24), f32)])

    partial = _sc_edge_pass(table, eij, n_pad, total_chunks)
    sums = _node_sc_pass(partial, t2, n_pad)
    res = _combine_pass(sums)
    return res[0, 0]
